# baseline (speedup 1.0000x reference)
import jax
import jax.numpy as jnp
from jax.experimental import pallas as pl
from jax.experimental.pallas import tpu as pltpu

_BM = 512
_NBUF = 3


def _outer_kernel(w_ref, adj_hbm, seq_hbm, out_hbm):
    n = adj_hbm.shape[0]
    d = seq_hbm.shape[1]
    w = w_ref[0, 0]

    def inner(adj_ref, seq_ref, out_ref):
        y = jnp.dot(
            adj_ref[...], seq_ref[...], preferred_element_type=jnp.float32
        )
        out_ref[...] = jnp.where(y >= 0, y, w * y)

    pipeline = pltpu.emit_pipeline(
        inner,
        grid=(n // _BM,),
        in_specs=[
            pl.BlockSpec(
                (_BM, n), lambda i: (i, 0), pipeline_mode=pl.Buffered(_NBUF)
            ),
            pl.BlockSpec((n, d), lambda i: (0, 0)),
        ],
        out_specs=[pl.BlockSpec((_BM, d), lambda i: (i, 0))],
    )
    pipeline(adj_hbm, seq_hbm, out_hbm)


def kernel(seq, adj_avg, prelu_weight):
    n, d = seq.shape
    w2d = prelu_weight.reshape(1, 1)
    return pl.pallas_call(
        _outer_kernel,
        in_specs=[
            pl.BlockSpec(memory_space=pltpu.SMEM),
            pl.BlockSpec(memory_space=pltpu.MemorySpace.HBM),
            pl.BlockSpec(memory_space=pltpu.MemorySpace.HBM),
        ],
        out_specs=pl.BlockSpec(memory_space=pltpu.MemorySpace.HBM),
        out_shape=jax.ShapeDtypeStruct((n, d), jnp.float32),
    )(w2d, adj_avg, seq)


# speedup vs baseline: 1.0714x; 1.0714x over previous
import jax
import jax.numpy as jnp
from jax.experimental import pallas as pl

_BM = 512


def _matmul_prelu_kernel(w_ref, adj_ref, seq_ref, out_ref):
    y = jnp.dot(adj_ref[...], seq_ref[...], preferred_element_type=jnp.float32)
    w = w_ref[0, 0]
    out_ref[...] = jnp.where(y >= 0, y, w * y)


def kernel(seq, adj_avg, prelu_weight):
    n, d = seq.shape
    w2d = prelu_weight.reshape(1, 1)
    grid = (n // _BM,)
    return pl.pallas_call(
        _matmul_prelu_kernel,
        grid=grid,
        in_specs=[
            pl.BlockSpec((1, 1), lambda i: (0, 0)),
            pl.BlockSpec((_BM, n), lambda i: (i, 0)),
            pl.BlockSpec((n, d), lambda i: (0, 0)),
        ],
        out_specs=pl.BlockSpec((_BM, d), lambda i: (i, 0)),
        out_shape=jax.ShapeDtypeStruct((n, d), jnp.float32),
    )(w2d, adj_avg, seq)
